# baseline (speedup 1.0000x reference)
import functools

import jax
import jax.numpy as jnp
from jax import lax
from jax.experimental import pallas as pl
from jax.experimental.pallas import tpu as pltpu
from jax.experimental.pallas import tpu_sc as plsc

NC = 2
NS = 16
NW = NC * NS
CH = 128
LANES = 16


def _sc_scores(B, K, D):
    RPW = B // NW
    NCH = RPW // CH
    NG = RPW // LANES
    mesh = plsc.VectorSubcoreMesh(
        core_axis_name="c", subcore_axis_name="s", num_cores=NC, num_subcores=NS
    )

    @functools.partial(
        pl.kernel,
        out_type=(
            jax.ShapeDtypeStruct((NW, NG, LANES), jnp.float32),
            jax.ShapeDtypeStruct((NW, K * NG, LANES), jnp.float32),
        ),
        mesh=mesh,
        compiler_params=pltpu.CompilerParams(needs_layout_passes=False, use_tc_tiling_on_sc=False),
        scratch_types=[
            pltpu.VMEM((NCH, CH), jnp.int32),
            pltpu.VMEM((NCH, CH), jnp.int32),
            pltpu.VMEM((K, NCH, CH), jnp.int32),
            pltpu.VMEM((RPW, D), jnp.float32),
            pltpu.VMEM((RPW, D), jnp.float32),
            pltpu.VMEM((RPW, D), jnp.float32),
            pltpu.VMEM((NG, LANES), jnp.float32),
            pltpu.VMEM((K * NG, LANES), jnp.float32),
            pltpu.SemaphoreType.DMA,
        ],
    )
    def sc_scores(ci_h, ni_h, gi_h, ine_h, oute_h, pos_h, neg_h,
                  idx_c, idx_n, idx_g, crows, nrows, gbuf, pos_v, neg_v, sem):
        wid = lax.axis_index("c") * NS + lax.axis_index("s")
        pltpu.sync_copy(ci_h.at[wid], idx_c)
        pltpu.sync_copy(ni_h.at[wid], idx_n)
        pltpu.sync_copy(gi_h.at[wid], idx_g)

        descs = []
        for ch in range(NCH):
            descs.append(pltpu.async_copy(
                ine_h.at[idx_c.at[ch]], crows.at[pl.ds(ch * CH, CH)], sem))
            descs.append(pltpu.async_copy(
                oute_h.at[idx_n.at[ch]], nrows.at[pl.ds(ch * CH, CH)], sem))
        for dd in descs:
            dd.wait()

        lanes = lax.iota(jnp.int32, LANES)

        def dot16(aref, bref, row16):
            acc = jnp.zeros((LANES,), jnp.float32)
            for d in range(D):
                col = jnp.full((LANES,), d, jnp.int32)
                acc = acc + (plsc.load_gather(aref, [row16, col])
                             * plsc.load_gather(bref, [row16, col]))
            return acc

        def pos_g(g, carry):
            row16 = lanes + g * LANES
            pos_v[g, :] = dot16(crows, nrows, row16)
            return carry

        lax.fori_loop(0, NG, pos_g, 0)

        def neg_k(k, carry):
            ds_ = [pltpu.async_copy(
                oute_h.at[idx_g.at[k, ch]], gbuf.at[pl.ds(ch * CH, CH)], sem)
                for ch in range(NCH)]
            for dd in ds_:
                dd.wait()

            def neg_g(g, c2):
                row16 = lanes + g * LANES
                neg_v[k * NG + g, :] = dot16(crows, gbuf, row16)
                return c2

            lax.fori_loop(0, NG, neg_g, 0)
            return carry

        lax.fori_loop(0, K, neg_k, 0)

        pltpu.sync_copy(pos_v, pos_h.at[wid])
        pltpu.sync_copy(neg_v, neg_h.at[wid])

    return sc_scores


def _logsig(x):
    return jnp.minimum(x, 0.0) - jnp.log1p(jnp.exp(-jnp.abs(x)))


def _tc_loss(p_ref, n_ref, o_ref):
    lp = jnp.sum(_logsig(p_ref[...]))
    ln = jnp.sum(_logsig(-n_ref[...]))
    o_ref[0, 0] = -(lp + ln)


def kernel(centre, neighbour, neg_samples, in_emb, out_emb):
    B = centre.shape[0]
    K = neg_samples.shape[1]
    D = in_emb.shape[1]
    RPW = B // NW
    NCH = RPW // CH

    ci = centre.astype(jnp.int32).reshape(NW, NCH, CH)
    ni = neighbour.astype(jnp.int32).reshape(NW, NCH, CH)
    gi = (neg_samples.astype(jnp.int32)
          .reshape(NW, RPW, K).transpose(0, 2, 1).reshape(NW, K, NCH, CH))

    pos, neg = _sc_scores(B, K, D)(ci, ni, gi, in_emb, out_emb)

    out = pl.pallas_call(
        _tc_loss,
        out_shape=jax.ShapeDtypeStruct((1, 1), jnp.float32),
        out_specs=pl.BlockSpec(memory_space=pltpu.SMEM),
    )(pos.reshape(B // 128, 128), neg.reshape(B * K // 128, 128))
    return out[0, 0]


# speedup vs baseline: 1.0581x; 1.0581x over previous
import functools

import jax
import jax.numpy as jnp
from jax import lax
from jax.experimental import pallas as pl
from jax.experimental.pallas import tpu as pltpu
from jax.experimental.pallas import tpu_sc as plsc

NC = 2
NS = 16
NW = NC * NS
CH = 128
LANES = 16
NBUF = 4


def _sc_scores(B, K, D):
    RPW = B // NW
    NCH = RPW // CH
    NGR = RPW // LANES
    FL = RPW * K
    NCHG = FL // CH
    GPC = CH // LANES
    mesh = plsc.VectorSubcoreMesh(
        core_axis_name="c", subcore_axis_name="s", num_cores=NC, num_subcores=NS
    )

    @functools.partial(
        pl.kernel,
        out_type=(
            jax.ShapeDtypeStruct((NW, NGR, LANES), jnp.float32),
            jax.ShapeDtypeStruct((NW, K * NGR, LANES), jnp.float32),
        ),
        mesh=mesh,
        compiler_params=pltpu.CompilerParams(
            needs_layout_passes=False, use_tc_tiling_on_sc=False),
        scratch_types=[
            pltpu.VMEM((NCH, CH), jnp.int32),
            pltpu.VMEM((NCH, CH), jnp.int32),
            pltpu.VMEM((NCHG, CH), jnp.int32),
            pltpu.VMEM((RPW, D), jnp.float32),
            pltpu.VMEM((RPW, D), jnp.float32),
            pltpu.VMEM((NBUF, CH, D), jnp.float32),
            pltpu.VMEM((NGR, LANES), jnp.float32),
            pltpu.VMEM((K * NGR, LANES), jnp.float32),
            pltpu.SemaphoreType.DMA,
            pltpu.SemaphoreType.DMA,
            [pltpu.SemaphoreType.DMA] * NBUF,
        ],
    )
    def sc_scores(ci_h, ni_h, gi_h, ine_h, oute_h, pos_h, neg_h,
                  idx_c, idx_n, idx_g, crows, nrows, gring, pos_v, neg_v,
                  sem_c, sem_n, gsems):
        wid = lax.axis_index("c") * NS + lax.axis_index("s")
        pltpu.sync_copy(ci_h.at[wid], idx_c)
        pltpu.sync_copy(ni_h.at[wid], idx_n)
        pltpu.sync_copy(gi_h.at[wid], idx_g)

        cds, nds = [], []
        for ch in range(NCH):
            cds.append(pltpu.async_copy(
                ine_h.at[idx_c.at[ch]], crows.at[pl.ds(ch * CH, CH)], sem_c))
            nds.append(pltpu.async_copy(
                oute_h.at[idx_n.at[ch]], nrows.at[pl.ds(ch * CH, CH)], sem_n))
        for b in range(NBUF):
            pltpu.async_copy(oute_h.at[idx_g.at[b]], gring.at[b], gsems[b])
        for dd in cds + nds:
            dd.wait()

        lanes = lax.iota(jnp.int32, LANES)

        def dot16(aref, arow16, bref, brow16):
            acc = jnp.zeros((LANES,), jnp.float32)
            for d in range(D):
                col = jnp.full((LANES,), d, jnp.int32)
                acc = acc + (plsc.load_gather(aref, [arow16, col])
                             * plsc.load_gather(bref, [brow16, col]))
            return acc

        def pos_g(g, carry):
            row16 = lanes + g * LANES
            pos_v[g, :] = dot16(crows, row16, nrows, row16)
            return carry

        lax.fori_loop(0, NGR, pos_g, 0)

        def neg_cc(cc, carry):
            for j in range(NBUF):
                c = cc * NBUF + j
                buf = gring.at[j]
                pltpu.make_async_copy(
                    oute_h.at[pl.ds(0, CH)], buf, gsems[j]).wait()
                base = c * CH

                def chunk_g(g, c2):
                    p16 = base + g * LANES + lanes
                    row16 = p16 // K
                    neg_v[c * GPC + g, :] = dot16(crows, row16, buf,
                                                  lanes + g * LANES)
                    return c2

                lax.fori_loop(0, GPC, chunk_g, 0)

                @pl.when(c + NBUF < NCHG)
                def _():
                    pltpu.async_copy(
                        oute_h.at[idx_g.at[c + NBUF]], buf, gsems[j])
            return carry

        lax.fori_loop(0, NCHG // NBUF, neg_cc, 0)

        pltpu.sync_copy(pos_v, pos_h.at[wid])
        pltpu.sync_copy(neg_v, neg_h.at[wid])

    return sc_scores


def _logsig(x):
    return jnp.minimum(x, 0.0) - jnp.log1p(jnp.exp(-jnp.abs(x)))


def _tc_loss(p_ref, n_ref, o_ref):
    lp = jnp.sum(_logsig(p_ref[...]))
    ln = jnp.sum(_logsig(-n_ref[...]))
    o_ref[0, 0] = -(lp + ln)


def kernel(centre, neighbour, neg_samples, in_emb, out_emb):
    B = centre.shape[0]
    K = neg_samples.shape[1]
    D = in_emb.shape[1]
    RPW = B // NW

    ci = centre.astype(jnp.int32).reshape(NW, RPW // CH, CH)
    ni = neighbour.astype(jnp.int32).reshape(NW, RPW // CH, CH)
    gi = neg_samples.astype(jnp.int32).reshape(NW, RPW * K // CH, CH)

    pos, neg = _sc_scores(B, K, D)(ci, ni, gi, in_emb, out_emb)

    out = pl.pallas_call(
        _tc_loss,
        out_shape=jax.ShapeDtypeStruct((1, 1), jnp.float32),
        out_specs=pl.BlockSpec(memory_space=pltpu.SMEM),
    )(pos.reshape(B // 128, 128), neg.reshape(B * K // 128, 128))
    return out[0, 0]
